# Initial kernel scaffold; baseline (speedup 1.0000x reference)
import jax
import jax.numpy as jnp
from jax.experimental import pallas as pl


def kernel(des, tweet, num_prop, cat_prop, new_feature, edge_index, edge_type, W_des, b_des, W_tw, b_tw, W_np, b_np, W_cp, b_cp, W_nf, b_nf, W_in, b_in, Wg1, bg1, Wg2, bg2, W_o1, b_o1, W_o2, b_o2):
    raise NotImplementedError("write your pallas kernel here")


# speedup vs baseline: 9.8298x; 9.8298x over previous
import functools

import jax
import jax.numpy as jnp
from jax import lax
from jax.experimental import pallas as pl
from jax.experimental.pallas import tpu as pltpu
from jax.experimental.pallas import tpu_sc as plsc

N = 10000
E = 320000
NC, NS = 2, 16
NW = NC * NS
EPW = E // NW
CH = 128
NCHUNK = -(-EPW // CH)
EPWP = NCHUNK * CH
NPAD = N + 8
RB = 80
NRB = N // RB
RB_ITERS = (NRB + NS - 1) // NS

_mesh = plsc.VectorSubcoreMesh(core_axis_name="c", subcore_axis_name="s",
                               num_cores=NC, num_subcores=NS)


def _leaky(x):
    return jnp.where(x > 0, x, 0.01 * x)


def _rows_loop(sid, fn):
    def body(k, _):
        j = jnp.minimum(sid + k * NS, NRB - 1)
        fn(pl.multiple_of(j * RB, RB))
        return 0
    lax.fori_loop(0, RB_ITERS, body, 0)


_DEG_OUT = jax.ShapeDtypeStruct((NC, N, 128), jnp.float32)
NBUFD = 4
_MAIND = (NCHUNK // NBUFD) * NBUFD
_DEG_SCRATCH = (
    [pltpu.VMEM((CH, 128), jnp.float32),
     pltpu.VMEM((RB, 128), jnp.float32)]
    + [pltpu.VMEM((CH,), jnp.int32)] * NBUFD
    + [pltpu.SemaphoreType.DMA] * (2 * NBUFD)
)


def _deg_body(dstp_hbm, ones_hbm, zero_hbm, out_hbm, ones_v, zero_v, *rest):
    dst_i = list(rest[:NBUFD])
    dsem, ssem = rest[NBUFD:2 * NBUFD], rest[2 * NBUFD:2 * NBUFD + NBUFD]
    acc_sh = rest[-1]
    cid = lax.axis_index("c")
    sid = lax.axis_index("s")
    wid = sid * NC + cid
    base = wid * EPWP
    pltpu.sync_copy(ones_hbm, ones_v)
    pltpu.sync_copy(zero_hbm, zero_v)
    _rows_loop(sid, lambda off: pltpu.sync_copy(zero_v, acc_sh.at[pl.ds(off, RB)]))
    plsc.subcore_barrier()

    def _idx(j, b):
        off = pl.multiple_of(base + j * CH, CH)
        return pltpu.async_copy(dstp_hbm.at[pl.ds(off, CH)], dst_i[b], dsem[b])

    def _scat(b):
        return pltpu.async_copy(ones_v, acc_sh.at[dst_i[b]], ssem[b], add=True)

    for b in range(NBUFD):
        _idx(b, b)

    def round_(r, _):
        for b in range(NBUFD):
            jc = r * NBUFD + b
            pltpu.make_async_copy(dstp_hbm.at[pl.ds(CH, CH)], dst_i[b],
                                  dsem[b]).wait()
            _scat(b).wait()
            _idx(jnp.minimum(jc + NBUFD, NCHUNK - 1), b)
        return 0
    lax.fori_loop(0, _MAIND // NBUFD, round_, 0)

    for b in range(NBUFD):
        jc = _MAIND + b
        pltpu.make_async_copy(dstp_hbm.at[pl.ds(CH, CH)], dst_i[b],
                              dsem[b]).wait()
        if jc < NCHUNK:
            pltpu.sync_copy(ones_v, acc_sh.at[dst_i[b]], add=True)
    plsc.subcore_barrier()

    def _read(off):
        pltpu.sync_copy(acc_sh.at[pl.ds(off, RB)], zero_v)
        pltpu.sync_copy(zero_v, out_hbm.at[cid, pl.ds(off, RB)])
    _rows_loop(sid, _read)


_CONV_OUT = jax.ShapeDtypeStruct((NC, N, 128), jnp.float32)
NBUF = 2
_MAIN = (NCHUNK // NBUF) * NBUF
_CONV_SCRATCH = (
    [pltpu.VMEM((NCHUNK, 1, CH), jnp.int32)]
    + [pltpu.VMEM((CH, 128), jnp.float32)] * NBUF
    + [pltpu.VMEM((CH,), jnp.int32)] * NBUF
    + [pltpu.SemaphoreType.DMA] * (3 * NBUF)
)


def _conv_body(hp_hbm, srcp_hbm, dstp_hbm, out_hbm, src_v, *rest):
    rows = list(rest[:NBUF])
    dst_i = list(rest[NBUF:2 * NBUF])
    gsem = rest[2 * NBUF:3 * NBUF]
    dsem = rest[3 * NBUF:4 * NBUF]
    ssem = rest[4 * NBUF:5 * NBUF]
    acc_sh = rest[-1]
    cid = lax.axis_index("c")
    sid = lax.axis_index("s")
    wid = sid * NC + cid
    base = wid * EPWP
    def _init(off):
        pltpu.sync_copy(hp_hbm.at[pl.ds(off, RB)], rows[0].at[pl.ds(0, RB)])
        pltpu.sync_copy(rows[0].at[pl.ds(0, RB)], acc_sh.at[pl.ds(off, RB)])
    _rows_loop(sid, _init)
    pltpu.sync_copy(srcp_hbm.at[pl.ds(wid * NCHUNK, NCHUNK)], src_v)
    plsc.subcore_barrier()

    def _idx(j, b):
        off = pl.multiple_of(base + j * CH, CH)
        return pltpu.async_copy(dstp_hbm.at[pl.ds(off, CH)], dst_i[b], dsem[b])

    def _gather(j, b):
        return pltpu.async_copy(hp_hbm.at[src_v.at[j, 0]], rows[b], gsem[b])

    for b in range(NBUF):
        _idx(b, b)
        _gather(b, b)

    def round_(r, _):
        for b in range(NBUF):
            jc = r * NBUF + b
            pltpu.make_async_copy(hp_hbm.at[src_v.at[jc, 0]], rows[b],
                                  gsem[b]).wait()
            pltpu.make_async_copy(dstp_hbm.at[pl.ds(CH, CH)], dst_i[b],
                                  dsem[b]).wait()
            pltpu.async_copy(rows[b], acc_sh.at[dst_i[b]],
                             ssem[b], add=True).wait()
            jn = jnp.minimum(jc + NBUF, NCHUNK - 1)
            _idx(jn, b)
            _gather(jn, b)
        return 0
    lax.fori_loop(0, _MAIN // NBUF, round_, 0)

    for b in range(NBUF):
        jc = _MAIN + b
        pltpu.make_async_copy(hp_hbm.at[src_v.at[min(jc, NCHUNK - 1), 0]],
                              rows[b], gsem[b]).wait()
        pltpu.make_async_copy(dstp_hbm.at[pl.ds(CH, CH)], dst_i[b],
                              dsem[b]).wait()
        if jc < NCHUNK:
            pltpu.sync_copy(rows[b], acc_sh.at[dst_i[b]], add=True)
    plsc.subcore_barrier()

    def _read(off):
        pltpu.sync_copy(acc_sh.at[pl.ds(off, RB)], rows[0].at[pl.ds(0, RB)])
        pltpu.sync_copy(rows[0].at[pl.ds(0, RB)],
                        out_hbm.at[cid, pl.ds(off, RB)])
    _rows_loop(sid, _read)


_deg = pl.kernel(_deg_body, out_type=_DEG_OUT, mesh=_mesh,
                 scratch_types=_DEG_SCRATCH
                 + [pltpu.VMEM_SHARED((NPAD, 128), jnp.float32)])
_conv = pl.kernel(_conv_body, out_type=_CONV_OUT, mesh=_mesh,
                  scratch_types=_CONV_SCRATCH
                  + [pltpu.VMEM_SHARED((NPAD, 128), jnp.float32)])


_BLK = 1000
_GRID = N // _BLK


def _row_spec(d):
    return pl.BlockSpec((_BLK, d), lambda i: (i, 0))


def _full_spec(shape):
    nd = len(shape)
    return pl.BlockSpec(shape, lambda i: (0,) * nd)


def _feat_body(des, tweet, sm, degp, Wd, Wt, Ws, bc, Win, bin_, Wg1,
               hp_out, dis_out):
    x = des[:] @ Wd[:] + tweet[:] @ Wt[:] + sm[:] @ Ws[:] + bc[:]
    x = _leaky(x)
    x = _leaky(x @ Win[:] + bin_[:])
    deg = degp[0, :, 0:8] + degp[1, :, 0:8] + 1.0
    dis8 = lax.rsqrt(deg)
    dis = dis8[:, 0:1]
    hp_out[:] = dis * (x @ Wg1[:])
    dis_out[:] = dis8


def _feat(des, tweet, sm, degp, Wd, Wt, Ws, bc, Win, bin_, Wg1):
    return pl.pallas_call(
        _feat_body,
        grid=(_GRID,),
        in_specs=[
            _row_spec(768), _row_spec(768), _row_spec(32),
            pl.BlockSpec((NC, _BLK, 128), lambda i: (0, i, 0)),
            _full_spec((768, 128)), _full_spec((768, 128)),
            _full_spec((32, 128)), _full_spec((1, 128)),
            _full_spec((128, 128)), _full_spec((1, 128)),
            _full_spec((128, 128)),
        ],
        out_specs=[_row_spec(128), _row_spec(8)],
        out_shape=[jax.ShapeDtypeStruct((N, 128), jnp.float32),
                   jax.ShapeDtypeStruct((N, 8), jnp.float32)],
    )(des, tweet, sm, degp, Wd, Wt, Ws, bc, Win, bin_, Wg1)


def _mid_body(accp, hp, dis8, Wg2, bg1, out):
    acc = accp[0] + accp[1] - hp[:]
    dis = dis8[:, 0:1]
    x2 = dis * acc + bg1[:]
    out[:] = dis * (x2 @ Wg2[:])


def _mid(accp, hp, dis8, Wg2, bg1):
    return pl.pallas_call(
        _mid_body,
        grid=(_GRID,),
        in_specs=[
            pl.BlockSpec((NC, _BLK, 128), lambda i: (0, i, 0)),
            _row_spec(128), _row_spec(8),
            _full_spec((128, 128)), _full_spec((1, 128)),
        ],
        out_specs=_row_spec(128),
        out_shape=jax.ShapeDtypeStruct((N, 128), jnp.float32),
    )(accp, hp, dis8, Wg2, bg1)


def _fin_body(accp, hp, dis8, bg2, Wo1, bo1, Wo2, bo2, out):
    acc = accp[0] + accp[1] - hp[:]
    dis = dis8[:, 0:1]
    x3 = dis * acc + bg2[:]
    y = _leaky(x3 @ Wo1[:] + bo1[:])
    out[:] = y @ Wo2[:] + bo2[:]


def _fin(accp, hp, dis8, bg2, Wo1, bo1, Wo2, bo2):
    return pl.pallas_call(
        _fin_body,
        grid=(_GRID,),
        in_specs=[
            pl.BlockSpec((NC, _BLK, 128), lambda i: (0, i, 0)),
            _row_spec(128), _row_spec(8),
            _full_spec((1, 128)),
            _full_spec((128, 128)), _full_spec((1, 128)),
            _full_spec((128, 2)), _full_spec((1, 2)),
        ],
        out_specs=_row_spec(2),
        out_shape=jax.ShapeDtypeStruct((N, 2), jnp.float32),
    )(accp, hp, dis8, bg2, Wo1, bo1, Wo2, bo2)


def kernel(des, tweet, num_prop, cat_prop, new_feature, edge_index, edge_type,
           W_des, b_des, W_tw, b_tw, W_np, b_np, W_cp, b_cp, W_nf, b_nf,
           W_in, b_in, Wg1, bg1, Wg2, bg2, W_o1, b_o1, W_o2, b_o2):
    f32 = jnp.float32
    sm = jnp.concatenate(
        [num_prop, cat_prop, new_feature, jnp.zeros((N, 13), f32)], axis=1)
    Wd = jnp.zeros((768, 128), f32).at[:, 0:28].set(W_des)
    Wt = jnp.zeros((768, 128), f32).at[:, 28:64].set(W_tw)
    Ws = (jnp.zeros((32, 128), f32)
          .at[0:7, 64:76].set(W_np)
          .at[7:18, 76:116].set(W_cp)
          .at[18:19, 116:128].set(W_nf))
    bc = jnp.concatenate([b_des, b_tw, b_np, b_cp, b_nf]).reshape(1, 128)

    pad = ((0, 0), (0, EPWP - EPW))
    srcp = jnp.pad(edge_index[0].reshape(NW, EPW), pad).reshape(-1)
    dstp = jnp.pad(edge_index[1].reshape(NW, EPW), pad,
                   constant_values=N).reshape(-1)
    src3d = srcp.reshape(NW * NCHUNK, 1, CH)

    ones_blk = jnp.ones((CH, 128), f32)
    zero_blk = jnp.zeros((RB, 128), f32)
    degp = _deg(dstp, ones_blk, zero_blk)
    hp1, dis8 = _feat(des, tweet, sm, degp, Wd, Wt, Ws, bc,
                      W_in, b_in.reshape(1, 128), Wg1)
    acc1 = _conv(hp1, src3d, dstp)
    hp2 = _mid(acc1, hp1, dis8, Wg2, bg1.reshape(1, 128))
    acc2 = _conv(hp2, src3d, dstp)
    return _fin(acc2, hp2, dis8, bg2.reshape(1, 128),
                W_o1, b_o1.reshape(1, 128), W_o2, b_o2.reshape(1, 2))


# speedup vs baseline: 10.0339x; 1.0208x over previous
import functools

import jax
import jax.numpy as jnp
from jax import lax
from jax.experimental import pallas as pl
from jax.experimental.pallas import tpu as pltpu
from jax.experimental.pallas import tpu_sc as plsc

N = 10000
E = 320000
NC, NS = 2, 16
NW = NC * NS
EPW = E // NW
CH = 128
NCHUNK = -(-EPW // CH)
EPWP = NCHUNK * CH
NPAD = N + 8
RB = 80
NRB = N // RB
RB_ITERS = (NRB + NS - 1) // NS

_mesh = plsc.VectorSubcoreMesh(core_axis_name="c", subcore_axis_name="s",
                               num_cores=NC, num_subcores=NS)


def _leaky(x):
    return jnp.where(x > 0, x, 0.01 * x)


def _rows_loop(sid, fn):
    def body(k, _):
        j = jnp.minimum(sid + k * NS, NRB - 1)
        fn(pl.multiple_of(j * RB, RB))
        return 0
    lax.fori_loop(0, RB_ITERS, body, 0)


_DEG_OUT = jax.ShapeDtypeStruct((NC, N, 128), jnp.float32)
NBUFD = 4
_MAIND = (NCHUNK // NBUFD) * NBUFD
_DEG_SCRATCH = (
    [pltpu.VMEM((CH, 128), jnp.float32),
     pltpu.VMEM((RB, 128), jnp.float32)]
    + [pltpu.VMEM((CH,), jnp.int32)] * NBUFD
    + [pltpu.SemaphoreType.DMA] * (2 * NBUFD)
)


def _deg_body(dstp_hbm, ones_hbm, zero_hbm, out_hbm, ones_v, zero_v, *rest):
    dst_i = list(rest[:NBUFD])
    dsem, ssem = rest[NBUFD:2 * NBUFD], rest[2 * NBUFD:2 * NBUFD + NBUFD]
    acc_sh = rest[-1]
    cid = lax.axis_index("c")
    sid = lax.axis_index("s")
    wid = sid * NC + cid
    base = wid * EPWP
    pltpu.sync_copy(ones_hbm, ones_v)
    pltpu.sync_copy(zero_hbm, zero_v)
    _rows_loop(sid, lambda off: pltpu.sync_copy(zero_v, acc_sh.at[pl.ds(off, RB)]))
    plsc.subcore_barrier()

    def _idx(j, b):
        off = pl.multiple_of(base + j * CH, CH)
        return pltpu.async_copy(dstp_hbm.at[pl.ds(off, CH)], dst_i[b], dsem[b])

    def _scat(b):
        return pltpu.async_copy(ones_v, acc_sh.at[dst_i[b]], ssem[b], add=True)

    for b in range(NBUFD):
        _idx(b, b)

    def round_(r, _):
        for b in range(NBUFD):
            jc = r * NBUFD + b
            pltpu.make_async_copy(dstp_hbm.at[pl.ds(CH, CH)], dst_i[b],
                                  dsem[b]).wait()
            _scat(b).wait()
            _idx(jnp.minimum(jc + NBUFD, NCHUNK - 1), b)
        return 0
    lax.fori_loop(0, _MAIND // NBUFD, round_, 0)

    for b in range(NBUFD):
        jc = _MAIND + b
        pltpu.make_async_copy(dstp_hbm.at[pl.ds(CH, CH)], dst_i[b],
                              dsem[b]).wait()
        if jc < NCHUNK:
            pltpu.sync_copy(ones_v, acc_sh.at[dst_i[b]], add=True)
    plsc.subcore_barrier()

    def _read(off):
        pltpu.sync_copy(acc_sh.at[pl.ds(off, RB)], zero_v)
        pltpu.sync_copy(zero_v, out_hbm.at[cid, pl.ds(off, RB)])
    _rows_loop(sid, _read)


_CONV_OUT = jax.ShapeDtypeStruct((NC, N, 128), jnp.float32)
NBUF = 2
_MAIN = (NCHUNK // NBUF) * NBUF
_CONV_SCRATCH = (
    [pltpu.VMEM((NCHUNK, 1, CH), jnp.int32)]
    + [pltpu.VMEM((CH, 128), jnp.float32)] * NBUF
    + [pltpu.VMEM((CH,), jnp.int32)] * NBUF
    + [pltpu.SemaphoreType.DMA] * (3 * NBUF)
)


def _conv_body(hp_hbm, srcp_hbm, dstp_hbm, out_hbm, src_v, *rest):
    rows = list(rest[:NBUF])
    dst_i = list(rest[NBUF:2 * NBUF])
    gsem = rest[2 * NBUF:3 * NBUF]
    dsem = rest[3 * NBUF:4 * NBUF]
    ssem = rest[4 * NBUF:5 * NBUF]
    acc_sh = rest[-1]
    cid = lax.axis_index("c")
    sid = lax.axis_index("s")
    wid = sid * NC + cid
    base = wid * EPWP
    def _init(off):
        pltpu.sync_copy(hp_hbm.at[pl.ds(off, RB)], rows[0].at[pl.ds(0, RB)])
        pltpu.sync_copy(rows[0].at[pl.ds(0, RB)], acc_sh.at[pl.ds(off, RB)])
    _rows_loop(sid, _init)
    pltpu.sync_copy(srcp_hbm.at[pl.ds(wid * NCHUNK, NCHUNK)], src_v)
    plsc.subcore_barrier()

    def _idx(j, b):
        off = pl.multiple_of(base + j * CH, CH)
        return pltpu.async_copy(dstp_hbm.at[pl.ds(off, CH)], dst_i[b], dsem[b])

    def _gather(j, b):
        return pltpu.async_copy(hp_hbm.at[src_v.at[j, 0]], rows[b], gsem[b])

    for b in range(NBUF):
        _idx(b, b)
        _gather(b, b)

    def round_(r, _):
        for b in range(NBUF):
            jc = r * NBUF + b
            pltpu.make_async_copy(hp_hbm.at[src_v.at[jc, 0]], rows[b],
                                  gsem[b]).wait()
            pltpu.make_async_copy(dstp_hbm.at[pl.ds(CH, CH)], dst_i[b],
                                  dsem[b]).wait()
            pltpu.async_copy(rows[b], acc_sh.at[dst_i[b]],
                             ssem[b], add=True).wait()
            jn = jnp.minimum(jc + NBUF, NCHUNK - 1)
            _idx(jn, b)
            _gather(jn, b)
        return 0
    lax.fori_loop(0, _MAIN // NBUF, round_, 0)

    for b in range(NBUF):
        jc = _MAIN + b
        pltpu.make_async_copy(hp_hbm.at[src_v.at[min(jc, NCHUNK - 1), 0]],
                              rows[b], gsem[b]).wait()
        pltpu.make_async_copy(dstp_hbm.at[pl.ds(CH, CH)], dst_i[b],
                              dsem[b]).wait()
        if jc < NCHUNK:
            pltpu.sync_copy(rows[b], acc_sh.at[dst_i[b]], add=True)
    plsc.subcore_barrier()

    def _read(off):
        pltpu.sync_copy(acc_sh.at[pl.ds(off, RB)], rows[0].at[pl.ds(0, RB)])
        pltpu.sync_copy(rows[0].at[pl.ds(0, RB)],
                        out_hbm.at[cid, pl.ds(off, RB)])
    _rows_loop(sid, _read)


_deg = pl.kernel(_deg_body, out_type=_DEG_OUT, mesh=_mesh,
                 scratch_types=_DEG_SCRATCH
                 + [pltpu.VMEM_SHARED((NPAD, 128), jnp.float32)])
_conv = pl.kernel(_conv_body, out_type=_CONV_OUT, mesh=_mesh,
                  scratch_types=_CONV_SCRATCH
                  + [pltpu.VMEM_SHARED((NPAD, 128), jnp.float32)])


_BLK = 1000
_GRID = N // _BLK


def _row_spec(d):
    return pl.BlockSpec((_BLK, d), lambda i: (i, 0))


def _full_spec(shape):
    nd = len(shape)
    return pl.BlockSpec(shape, lambda i: (0,) * nd)


def _featA_body(des, tweet, sm, Wd, Wt, Ws, bc, Win, bin_, x_out):
    x = des[:] @ Wd[:] + tweet[:] @ Wt[:] + sm[:] @ Ws[:] + bc[:]
    x = _leaky(x)
    x_out[:] = _leaky(x @ Win[:] + bin_[:])


def _featA(des, tweet, sm, Wd, Wt, Ws, bc, Win, bin_):
    return pl.pallas_call(
        _featA_body,
        grid=(_GRID,),
        in_specs=[
            _row_spec(768), _row_spec(768), _row_spec(32),
            _full_spec((768, 128)), _full_spec((768, 128)),
            _full_spec((32, 128)), _full_spec((1, 128)),
            _full_spec((128, 128)), _full_spec((1, 128)),
        ],
        out_specs=_row_spec(128),
        out_shape=jax.ShapeDtypeStruct((N, 128), jnp.float32),
    )(des, tweet, sm, Wd, Wt, Ws, bc, Win, bin_)


def _featB_body(x, degp, Wg1, hp_out, dis_out):
    deg = degp[0, :, 0:8] + degp[1, :, 0:8] + 1.0
    dis8 = lax.rsqrt(deg)
    dis = dis8[:, 0:1]
    hp_out[:] = dis * (x[:] @ Wg1[:])
    dis_out[:] = dis8


def _featB(x, degp, Wg1):
    return pl.pallas_call(
        _featB_body,
        grid=(_GRID,),
        in_specs=[
            _row_spec(128),
            pl.BlockSpec((NC, _BLK, 128), lambda i: (0, i, 0)),
            _full_spec((128, 128)),
        ],
        out_specs=[_row_spec(128), _row_spec(8)],
        out_shape=[jax.ShapeDtypeStruct((N, 128), jnp.float32),
                   jax.ShapeDtypeStruct((N, 8), jnp.float32)],
    )(x, degp, Wg1)


def _mid_body(accp, hp, dis8, Wg2, bg1, out):
    acc = accp[0] + accp[1] - hp[:]
    dis = dis8[:, 0:1]
    x2 = dis * acc + bg1[:]
    out[:] = dis * (x2 @ Wg2[:])


def _mid(accp, hp, dis8, Wg2, bg1):
    return pl.pallas_call(
        _mid_body,
        grid=(_GRID,),
        in_specs=[
            pl.BlockSpec((NC, _BLK, 128), lambda i: (0, i, 0)),
            _row_spec(128), _row_spec(8),
            _full_spec((128, 128)), _full_spec((1, 128)),
        ],
        out_specs=_row_spec(128),
        out_shape=jax.ShapeDtypeStruct((N, 128), jnp.float32),
    )(accp, hp, dis8, Wg2, bg1)


def _fin_body(accp, hp, dis8, bg2, Wo1, bo1, Wo2, bo2, out):
    acc = accp[0] + accp[1] - hp[:]
    dis = dis8[:, 0:1]
    x3 = dis * acc + bg2[:]
    y = _leaky(x3 @ Wo1[:] + bo1[:])
    out[:] = y @ Wo2[:] + bo2[:]


def _fin(accp, hp, dis8, bg2, Wo1, bo1, Wo2, bo2):
    return pl.pallas_call(
        _fin_body,
        grid=(_GRID,),
        in_specs=[
            pl.BlockSpec((NC, _BLK, 128), lambda i: (0, i, 0)),
            _row_spec(128), _row_spec(8),
            _full_spec((1, 128)),
            _full_spec((128, 128)), _full_spec((1, 128)),
            _full_spec((128, 2)), _full_spec((1, 2)),
        ],
        out_specs=_row_spec(2),
        out_shape=jax.ShapeDtypeStruct((N, 2), jnp.float32),
    )(accp, hp, dis8, bg2, Wo1, bo1, Wo2, bo2)


def kernel(des, tweet, num_prop, cat_prop, new_feature, edge_index, edge_type,
           W_des, b_des, W_tw, b_tw, W_np, b_np, W_cp, b_cp, W_nf, b_nf,
           W_in, b_in, Wg1, bg1, Wg2, bg2, W_o1, b_o1, W_o2, b_o2):
    f32 = jnp.float32
    sm = jnp.concatenate(
        [num_prop, cat_prop, new_feature, jnp.zeros((N, 13), f32)], axis=1)
    Wd = jnp.zeros((768, 128), f32).at[:, 0:28].set(W_des)
    Wt = jnp.zeros((768, 128), f32).at[:, 28:64].set(W_tw)
    Ws = (jnp.zeros((32, 128), f32)
          .at[0:7, 64:76].set(W_np)
          .at[7:18, 76:116].set(W_cp)
          .at[18:19, 116:128].set(W_nf))
    bc = jnp.concatenate([b_des, b_tw, b_np, b_cp, b_nf]).reshape(1, 128)

    pad = ((0, 0), (0, EPWP - EPW))
    srcp = jnp.pad(edge_index[0].reshape(NW, EPW), pad).reshape(-1)
    dstp = jnp.pad(edge_index[1].reshape(NW, EPW), pad,
                   constant_values=N).reshape(-1)
    src3d = srcp.reshape(NW * NCHUNK, 1, CH)

    ones_blk = jnp.ones((CH, 128), f32)
    zero_blk = jnp.zeros((RB, 128), f32)
    degp = _deg(dstp, ones_blk, zero_blk)
    x = _featA(des, tweet, sm, Wd, Wt, Ws, bc, W_in, b_in.reshape(1, 128))
    hp1, dis8 = _featB(x, degp, Wg1)
    acc1 = _conv(hp1, src3d, dstp)
    hp2 = _mid(acc1, hp1, dis8, Wg2, bg1.reshape(1, 128))
    acc2 = _conv(hp2, src3d, dstp)
    return _fin(acc2, hp2, dis8, bg2.reshape(1, 128),
                W_o1, b_o1.reshape(1, 128), W_o2, b_o2.reshape(1, 2))
